# baseline (speedup 1.0000x reference)
import functools

import jax
import jax.numpy as jnp
from jax import lax
from jax.experimental import pallas as pl
from jax.experimental.pallas import tpu as pltpu
from jax.experimental.pallas import tpu_sc as plsc

_B, _T, _DIN, _DSAE, _K = 16, 2, 768, 65536, 128
_BLK = 4096
_NBLK = _DSAE // _BLK
_MININT = -2147483648
_HALF = _DSAE // 2
_NV = _HALF // 16


def _mm_body(x_ref, w_ref, b_ref, key_ref, rm_ref):
    pre0 = jnp.dot(x_ref[0], w_ref[0], preferred_element_type=jnp.float32)
    pre1 = jnp.dot(x_ref[1], w_ref[1], preferred_element_type=jnp.float32)
    psum = pre0 + pre1 + b_ref[...]
    pb = lax.bitcast_convert_type(psum, jnp.int32)
    key_ref[...] = jnp.where(
        pb < 0, jnp.bitwise_xor(jnp.bitwise_not(pb), jnp.int32(_MININT)), pb)
    rm_ref[...] = 0.5 * (jnp.maximum(pre0, 0.0) + jnp.maximum(pre1, 0.0))


def _matmul_stage(xt, W_enc, b2):
    return pl.pallas_call(
        _mm_body,
        grid=(_NBLK,),
        in_specs=[
            pl.BlockSpec((_T, _B, _DIN), lambda i: (0, 0, 0)),
            pl.BlockSpec((_T, _DIN, _BLK), lambda i: (0, 0, i)),
            pl.BlockSpec((1, _BLK), lambda i: (0, i)),
        ],
        out_specs=[
            pl.BlockSpec((_B, _BLK), lambda i: (0, i)),
            pl.BlockSpec((_B, _BLK), lambda i: (0, i)),
        ],
        out_shape=[
            jax.ShapeDtypeStruct((_B, _DSAE), jnp.int32),
            jax.ShapeDtypeStruct((_B, _DSAE), jnp.float32),
        ],
        compiler_params=pltpu.CompilerParams(
            dimension_semantics=("arbitrary",),
        ),
    )(xt, W_enc, b2)


def _sc_body(key_hbm, rm_hbm, out_hbm, keys_v, rm_v, hist_v,
             loc_v, pair_v, shared, sem, sem2):
    c = lax.axis_index("c")
    s = lax.axis_index("s")
    row = c * 8 + s // 2
    half = s % 2
    col0 = half * _HALF

    keys_cp = pltpu.async_copy(key_hbm.at[row, pl.ds(col0, _HALF)], keys_v,
                               sem2)
    rm_cp = pltpu.async_copy(rm_hbm.at[row, pl.ds(col0, _HALF)], rm_v, sem)

    iota = lax.iota(jnp.int32, 16)
    lane_base = iota * 256
    ones = jnp.ones((16,), jnp.int32)
    _G = 8
    _VPG = _NV // _G

    def hist_pass(p, sh, prefix, wait=None):
        @plsc.parallel_loop(0, _G * 256, unroll=4)
        def _zero(i):
            hist_v[pl.ds(i * 16, 16)] = jnp.zeros((16,), jnp.int32)

        if wait is not None:
            wait.wait()

        def body(i, carry):
            @plsc.parallel_loop(0, _G, unroll=_G)
            def _g(g):
                k = keys_v[pl.ds((g * _VPG + i) * 16, 16)]
                if sh == 24:
                    bucket = lax.shift_right_arithmetic(k, 24) + 128
                    inc = ones
                else:
                    bucket = jnp.bitwise_and(
                        lax.shift_right_arithmetic(k, sh), 255)
                    m = lax.shift_right_arithmetic(k, sh + 8) == prefix
                    inc = m.astype(jnp.int32)
                idx = g * 4096 + lane_base + bucket
                cur = plsc.load_gather(hist_v, [idx])
                plsc.store_scatter(hist_v, [idx], cur + inc)
            return carry
        lax.fori_loop(0, _VPG, body, 0)

        @plsc.parallel_loop(0, 16, unroll=2)
        def _merge(j):
            acc = jnp.zeros((16,), jnp.int32)
            for g in range(_G):
                for lane in range(16):
                    acc = acc + hist_v[pl.ds(g * 4096 + lane * 256 + j * 16,
                                             16)]
            loc_v[pl.ds(j * 16, 16)] = acc

        exchange(p)

    def exchange(p):
        pltpu.sync_copy(loc_v, shared.at[p, s])
        plsc.subcore_barrier()
        pltpu.sync_copy(shared.at[p, jnp.bitwise_xor(s, 1)], pair_v)

        @plsc.parallel_loop(0, 16, unroll=4)
        def _pair(j):
            loc_v[pl.ds(j * 16, 16)] = (loc_v[pl.ds(j * 16, 16)]
                                        + pair_v[pl.ds(j * 16, 16)])

    def navigate(kk):
        def body(jj, carry):
            cum, cbkt, above, found = carry
            j = 15 - jj
            v = loc_v[pl.ds(j * 16, 16)]
            rev = lax.rev(v, (0,))
            cs = jnp.cumsum(rev)
            tot = jnp.sum(v)
            hit = jnp.logical_and(found == 0, cum + tot >= kk)
            i_rev = jnp.sum((cum + cs < kk).astype(jnp.int32))
            above_in = jnp.sum(jnp.where(iota == i_rev - 1, cs, 0))
            cbkt = jnp.where(hit, j * 16 + 15 - i_rev, cbkt)
            above = jnp.where(hit, cum + above_in, above)
            found = jnp.where(hit, jnp.int32(1), found)
            return (cum + tot, cbkt, above, found)
        _, cbkt, above, _ = lax.fori_loop(
            0, 16, body,
            (jnp.int32(0), jnp.int32(0), jnp.int32(0), jnp.int32(0)))
        return cbkt, above

    kk = jnp.int32(_K)
    with jax.named_scope("sc_p1"):
        hist_pass(0, 24, None, wait=keys_cp)
        c1, above = navigate(kk)
    kk = kk - above
    prefix = c1 - 128

    with jax.named_scope("sc_p2"):
        hist_pass(1, 16, prefix)
        c2, above = navigate(kk)
    kk = kk - above
    prefix = prefix * 256 + c2

    with jax.named_scope("sc_p3"):
        hist_pass(2, 8, prefix)
        c3, above = navigate(kk)
    kk = kk - above
    prefix = prefix * 256 + c3

    with jax.named_scope("sc_p4"):
        hist_pass(3, 0, prefix)
        c4, _ = navigate(kk)
    thr = prefix * 256 + c4

    rm_cp.wait()

    with jax.named_scope("sc_apply"):
        _Q = _NV // 4
        out_cps = []
        for q in range(4):
            @plsc.parallel_loop(q * _Q, (q + 1) * _Q, unroll=4)
            def _apply(i):
                k = keys_v[pl.ds(i * 16, 16)]
                r = rm_v[pl.ds(i * 16, 16)]
                rm_v[pl.ds(i * 16, 16)] = jnp.where(k >= thr, r,
                                                    jnp.float32(0.0))
            out_cps.append(pltpu.async_copy(
                rm_v.at[pl.ds(q * _Q * 16, _Q * 16)],
                out_hbm.at[row, pl.ds(col0 + q * _Q * 16, _Q * 16)], sem2))
        for cp in out_cps:
            cp.wait()


def _topk_mask_stage(key, rm):
    mesh = plsc.VectorSubcoreMesh(core_axis_name="c", subcore_axis_name="s")
    f = functools.partial(
        pl.kernel,
        out_type=jax.ShapeDtypeStruct((_B, _DSAE), jnp.float32),
        mesh=mesh,
        scratch_types=[
            pltpu.VMEM((_HALF,), jnp.int32),
            pltpu.VMEM((_HALF,), jnp.float32),
            pltpu.VMEM((8 * 16 * 256,), jnp.int32),
            pltpu.VMEM((256,), jnp.int32),
            pltpu.VMEM((256,), jnp.int32),
            pltpu.VMEM_SHARED((4, 16, 256), jnp.int32),
            pltpu.SemaphoreType.DMA,
            pltpu.SemaphoreType.DMA,
        ],
        compiler_params=pltpu.CompilerParams(needs_layout_passes=False),
    )(_sc_body)
    return f(key, rm)


def kernel(x, W_enc, b_enc):
    xt = jnp.transpose(x, (1, 0, 2))
    b2 = b_enc.reshape(1, _DSAE)
    key, rm = _matmul_stage(xt, W_enc, b2)
    return _topk_mask_stage(key, rm)


# speedup vs baseline: 1.0329x; 1.0329x over previous
import functools

import jax
import jax.numpy as jnp
from jax import lax
from jax.experimental import pallas as pl
from jax.experimental.pallas import tpu as pltpu
from jax.experimental.pallas import tpu_sc as plsc

_B, _T, _DIN, _DSAE, _K = 16, 2, 768, 65536, 128
_BLK = 4096
_NBLK = _DSAE // _BLK
_MININT = -2147483648
_HALF = _DSAE // 2
_NV = _HALF // 16


def _mm_body(x_ref, w_ref, b_ref, key_ref, rm_ref):
    pre0 = jnp.dot(x_ref[0], w_ref[0], preferred_element_type=jnp.float32)
    pre1 = jnp.dot(x_ref[1], w_ref[1], preferred_element_type=jnp.float32)
    psum = pre0 + pre1 + b_ref[...]
    pb = lax.bitcast_convert_type(psum, jnp.int32)
    key_ref[...] = jnp.where(
        pb < 0, jnp.bitwise_xor(jnp.bitwise_not(pb), jnp.int32(_MININT)), pb)
    rm_ref[...] = 0.5 * (jnp.maximum(pre0, 0.0) + jnp.maximum(pre1, 0.0))


def _matmul_stage(xt, W_enc, b2):
    return pl.pallas_call(
        _mm_body,
        grid=(_NBLK,),
        in_specs=[
            pl.BlockSpec((_T, _B, _DIN), lambda i: (0, 0, 0)),
            pl.BlockSpec((_T, _DIN, _BLK), lambda i: (0, 0, i)),
            pl.BlockSpec((1, _BLK), lambda i: (0, i)),
        ],
        out_specs=[
            pl.BlockSpec((_B, _BLK), lambda i: (0, i)),
            pl.BlockSpec((_B, _BLK), lambda i: (0, i)),
        ],
        out_shape=[
            jax.ShapeDtypeStruct((_B, _DSAE), jnp.int32),
            jax.ShapeDtypeStruct((_B, _DSAE), jnp.float32),
        ],
        compiler_params=pltpu.CompilerParams(
            dimension_semantics=("arbitrary",),
        ),
    )(xt, W_enc, b2)


def _sc_body(key_hbm, rm_hbm, out_hbm, keys_v, rm_v, hist_v,
             loc_v, pair_v, shared, sem, sem2):
    c = lax.axis_index("c")
    s = lax.axis_index("s")
    row = c * 8 + s // 2
    half = s % 2
    col0 = half * _HALF

    keys_cp = pltpu.async_copy(key_hbm.at[row, pl.ds(col0, _HALF)], keys_v,
                               sem2)
    rm_cp = pltpu.async_copy(rm_hbm.at[row, pl.ds(col0, _HALF)], rm_v, sem)

    iota = lax.iota(jnp.int32, 16)
    lane_base = iota * 256
    ones = jnp.ones((16,), jnp.int32)
    _G = 8
    _VPG = _NV // _G

    def hist_pass(p, sh, prefix, wait=None):
        @plsc.parallel_loop(0, _G * 256, unroll=4)
        def _zero(i):
            hist_v[pl.ds(i * 16, 16)] = jnp.zeros((16,), jnp.int32)

        if wait is not None:
            wait.wait()

        def body(i, carry):
            @plsc.parallel_loop(0, _G, unroll=_G)
            def _g(g):
                k = keys_v[pl.ds((g * _VPG + i) * 16, 16)]
                if sh == 24:
                    bucket = lax.shift_right_arithmetic(k, 24) + 128
                    inc = ones
                else:
                    bucket = jnp.bitwise_and(
                        lax.shift_right_arithmetic(k, sh), 255)
                    m = lax.shift_right_arithmetic(k, sh + 8) == prefix
                    inc = m.astype(jnp.int32)
                idx = g * 4096 + lane_base + bucket
                plsc.addupdate_scatter(hist_v, [idx], inc)
            return carry
        lax.fori_loop(0, _VPG, body, 0)

        @plsc.parallel_loop(0, 16, unroll=2)
        def _merge(j):
            acc = jnp.zeros((16,), jnp.int32)
            for g in range(_G):
                for lane in range(16):
                    acc = acc + hist_v[pl.ds(g * 4096 + lane * 256 + j * 16,
                                             16)]
            loc_v[pl.ds(j * 16, 16)] = acc

        exchange(p)

    def exchange(p):
        pltpu.sync_copy(loc_v, shared.at[p, s])
        plsc.subcore_barrier()
        pltpu.sync_copy(shared.at[p, jnp.bitwise_xor(s, 1)], pair_v)

        @plsc.parallel_loop(0, 16, unroll=4)
        def _pair(j):
            loc_v[pl.ds(j * 16, 16)] = (loc_v[pl.ds(j * 16, 16)]
                                        + pair_v[pl.ds(j * 16, 16)])

    def navigate(kk):
        def body(jj, carry):
            cum, cbkt, above, found = carry
            j = 15 - jj
            v = loc_v[pl.ds(j * 16, 16)]
            rev = lax.rev(v, (0,))
            cs = jnp.cumsum(rev)
            tot = jnp.sum(v)
            hit = jnp.logical_and(found == 0, cum + tot >= kk)
            i_rev = jnp.sum((cum + cs < kk).astype(jnp.int32))
            above_in = jnp.sum(jnp.where(iota == i_rev - 1, cs, 0))
            cbkt = jnp.where(hit, j * 16 + 15 - i_rev, cbkt)
            above = jnp.where(hit, cum + above_in, above)
            found = jnp.where(hit, jnp.int32(1), found)
            return (cum + tot, cbkt, above, found)
        _, cbkt, above, _ = lax.fori_loop(
            0, 16, body,
            (jnp.int32(0), jnp.int32(0), jnp.int32(0), jnp.int32(0)))
        return cbkt, above

    kk = jnp.int32(_K)
    with jax.named_scope("sc_p1"):
        hist_pass(0, 24, None, wait=keys_cp)
        c1, above = navigate(kk)
    kk = kk - above
    prefix = c1 - 128

    with jax.named_scope("sc_p2"):
        hist_pass(1, 16, prefix)
        c2, above = navigate(kk)
    kk = kk - above
    prefix = prefix * 256 + c2

    with jax.named_scope("sc_p3"):
        hist_pass(2, 8, prefix)
        c3, above = navigate(kk)
    kk = kk - above
    prefix = prefix * 256 + c3

    with jax.named_scope("sc_p4"):
        hist_pass(3, 0, prefix)
        c4, _ = navigate(kk)
    thr = prefix * 256 + c4

    rm_cp.wait()

    with jax.named_scope("sc_apply"):
        _Q = _NV // 4
        out_cps = []
        for q in range(4):
            @plsc.parallel_loop(q * _Q, (q + 1) * _Q, unroll=4)
            def _apply(i):
                k = keys_v[pl.ds(i * 16, 16)]
                r = rm_v[pl.ds(i * 16, 16)]
                rm_v[pl.ds(i * 16, 16)] = jnp.where(k >= thr, r,
                                                    jnp.float32(0.0))
            out_cps.append(pltpu.async_copy(
                rm_v.at[pl.ds(q * _Q * 16, _Q * 16)],
                out_hbm.at[row, pl.ds(col0 + q * _Q * 16, _Q * 16)], sem2))
        for cp in out_cps:
            cp.wait()


def _topk_mask_stage(key, rm):
    mesh = plsc.VectorSubcoreMesh(core_axis_name="c", subcore_axis_name="s")
    f = functools.partial(
        pl.kernel,
        out_type=jax.ShapeDtypeStruct((_B, _DSAE), jnp.float32),
        mesh=mesh,
        scratch_types=[
            pltpu.VMEM((_HALF,), jnp.int32),
            pltpu.VMEM((_HALF,), jnp.float32),
            pltpu.VMEM((8 * 16 * 256,), jnp.int32),
            pltpu.VMEM((256,), jnp.int32),
            pltpu.VMEM((256,), jnp.int32),
            pltpu.VMEM_SHARED((4, 16, 256), jnp.int32),
            pltpu.SemaphoreType.DMA,
            pltpu.SemaphoreType.DMA,
        ],
        compiler_params=pltpu.CompilerParams(needs_layout_passes=False),
    )(_sc_body)
    return f(key, rm)


def kernel(x, W_enc, b_enc):
    xt = jnp.transpose(x, (1, 0, 2))
    b2 = b_enc.reshape(1, _DSAE)
    key, rm = _matmul_stage(xt, W_enc, b2)
    return _topk_mask_stage(key, rm)


# speedup vs baseline: 1.0885x; 1.0538x over previous
import functools

import jax
import jax.numpy as jnp
from jax import lax
from jax.experimental import pallas as pl
from jax.experimental.pallas import tpu as pltpu
from jax.experimental.pallas import tpu_sc as plsc

_B, _T, _DIN, _DSAE, _K = 16, 2, 768, 65536, 128
_BLK = 4096
_NBLK = _DSAE // _BLK
_MININT = -2147483648
_HALF = _DSAE // 2
_NV = _HALF // 16


def _mm_body(x_ref, w_ref, b_ref, key_ref, rm_ref):
    pre0 = jnp.dot(x_ref[0], w_ref[0], preferred_element_type=jnp.float32)
    pre1 = jnp.dot(x_ref[1], w_ref[1], preferred_element_type=jnp.float32)
    psum = pre0 + pre1 + b_ref[...]
    pb = lax.bitcast_convert_type(psum, jnp.int32)
    key_ref[...] = jnp.where(
        pb < 0, jnp.bitwise_xor(jnp.bitwise_not(pb), jnp.int32(_MININT)), pb)
    rm_ref[...] = 0.5 * (jnp.maximum(pre0, 0.0) + jnp.maximum(pre1, 0.0))


def _matmul_stage(xt, W_enc, b2):
    return pl.pallas_call(
        _mm_body,
        grid=(_NBLK,),
        in_specs=[
            pl.BlockSpec((_T, _B, _DIN), lambda i: (0, 0, 0)),
            pl.BlockSpec((_T, _DIN, _BLK), lambda i: (0, 0, i)),
            pl.BlockSpec((1, _BLK), lambda i: (0, i)),
        ],
        out_specs=[
            pl.BlockSpec((_B, _BLK), lambda i: (0, i)),
            pl.BlockSpec((_B, _BLK), lambda i: (0, i)),
        ],
        out_shape=[
            jax.ShapeDtypeStruct((_B, _DSAE), jnp.int32),
            jax.ShapeDtypeStruct((_B, _DSAE), jnp.float32),
        ],
        compiler_params=pltpu.CompilerParams(
            dimension_semantics=("arbitrary",),
        ),
    )(xt, W_enc, b2)


def _sc_body(key_hbm, rm_hbm, out_hbm, keys_v, rm_v, hist_v,
             loc_v, pair_v, shared, sem, sem2):
    c = lax.axis_index("c")
    s = lax.axis_index("s")
    row = c * 8 + s // 2
    half = s % 2
    col0 = half * _HALF

    keys_cp = pltpu.async_copy(key_hbm.at[row, pl.ds(col0, _HALF)], keys_v,
                               sem2)
    rm_cp = pltpu.async_copy(rm_hbm.at[row, pl.ds(col0, _HALF)], rm_v, sem)

    iota = lax.iota(jnp.int32, 16)
    lane_base = iota * 256
    ones = jnp.ones((16,), jnp.int32)
    _G = 8
    _VPG = _NV // _G

    def hist_pass(p, sh, prefix, wait=None):
        if p == 0:
            @plsc.parallel_loop(0, _G * 256, unroll=4)
            def _zero(i):
                hist_v[pl.ds(i * 16, 16)] = jnp.zeros((16,), jnp.int32)

        if wait is not None:
            wait.wait()

        def body(i, carry):
            @plsc.parallel_loop(0, _G, unroll=_G)
            def _g(g):
                k = keys_v[pl.ds((g * _VPG + i) * 16, 16)]
                if sh == 24:
                    bucket = lax.shift_right_arithmetic(k, 24) + 128
                    inc = ones
                else:
                    bucket = jnp.bitwise_and(
                        lax.shift_right_arithmetic(k, sh), 255)
                    m = lax.shift_right_arithmetic(k, sh + 8) == prefix
                    inc = m.astype(jnp.int32)
                idx = g * 4096 + lane_base + bucket
                plsc.addupdate_scatter(hist_v, [idx], inc)
            return carry
        lax.fori_loop(0, _VPG, body, 0)

        zvec = jnp.zeros((16,), jnp.int32)

        @plsc.parallel_loop(0, 16, unroll=2)
        def _merge(j):
            acc = zvec
            for g in range(_G):
                for lane in range(16):
                    sl = pl.ds(g * 4096 + lane * 256 + j * 16, 16)
                    acc = acc + hist_v[sl]
                    hist_v[sl] = zvec
            loc_v[pl.ds(j * 16, 16)] = acc

        exchange(p)

    def exchange(p):
        pltpu.sync_copy(loc_v, shared.at[p, s])
        plsc.subcore_barrier()
        pltpu.sync_copy(shared.at[p, jnp.bitwise_xor(s, 1)], pair_v)

        @plsc.parallel_loop(0, 16, unroll=4)
        def _pair(j):
            loc_v[pl.ds(j * 16, 16)] = (loc_v[pl.ds(j * 16, 16)]
                                        + pair_v[pl.ds(j * 16, 16)])

    def navigate(kk):
        def body(jj, carry):
            cum, cbkt, above, found = carry
            j = 15 - jj
            v = loc_v[pl.ds(j * 16, 16)]
            rev = lax.rev(v, (0,))
            cs = jnp.cumsum(rev)
            tot = jnp.sum(v)
            hit = jnp.logical_and(found == 0, cum + tot >= kk)
            i_rev = jnp.sum((cum + cs < kk).astype(jnp.int32))
            above_in = jnp.sum(jnp.where(iota == i_rev - 1, cs, 0))
            cbkt = jnp.where(hit, j * 16 + 15 - i_rev, cbkt)
            above = jnp.where(hit, cum + above_in, above)
            found = jnp.where(hit, jnp.int32(1), found)
            return (cum + tot, cbkt, above, found)
        _, cbkt, above, _ = lax.fori_loop(
            0, 16, body,
            (jnp.int32(0), jnp.int32(0), jnp.int32(0), jnp.int32(0)))
        return cbkt, above

    kk = jnp.int32(_K)
    with jax.named_scope("sc_p1"):
        hist_pass(0, 24, None, wait=keys_cp)
        c1, above = navigate(kk)
    kk = kk - above
    prefix = c1 - 128

    with jax.named_scope("sc_p2"):
        hist_pass(1, 16, prefix)
        c2, above = navigate(kk)
    kk = kk - above
    prefix = prefix * 256 + c2

    with jax.named_scope("sc_p3"):
        hist_pass(2, 8, prefix)
        c3, above = navigate(kk)
    kk = kk - above
    prefix = prefix * 256 + c3

    with jax.named_scope("sc_p4"):
        hist_pass(3, 0, prefix)
        c4, _ = navigate(kk)
    thr = prefix * 256 + c4

    rm_cp.wait()

    with jax.named_scope("sc_apply"):
        _Q = _NV // 4
        out_cps = []
        for q in range(4):
            @plsc.parallel_loop(q * _Q, (q + 1) * _Q, unroll=4)
            def _apply(i):
                k = keys_v[pl.ds(i * 16, 16)]
                r = rm_v[pl.ds(i * 16, 16)]
                rm_v[pl.ds(i * 16, 16)] = jnp.where(k >= thr, r,
                                                    jnp.float32(0.0))
            out_cps.append(pltpu.async_copy(
                rm_v.at[pl.ds(q * _Q * 16, _Q * 16)],
                out_hbm.at[row, pl.ds(col0 + q * _Q * 16, _Q * 16)], sem2))
        for cp in out_cps:
            cp.wait()


def _topk_mask_stage(key, rm):
    mesh = plsc.VectorSubcoreMesh(core_axis_name="c", subcore_axis_name="s")
    f = functools.partial(
        pl.kernel,
        out_type=jax.ShapeDtypeStruct((_B, _DSAE), jnp.float32),
        mesh=mesh,
        scratch_types=[
            pltpu.VMEM((_HALF,), jnp.int32),
            pltpu.VMEM((_HALF,), jnp.float32),
            pltpu.VMEM((8 * 16 * 256,), jnp.int32),
            pltpu.VMEM((256,), jnp.int32),
            pltpu.VMEM((256,), jnp.int32),
            pltpu.VMEM_SHARED((4, 16, 256), jnp.int32),
            pltpu.SemaphoreType.DMA,
            pltpu.SemaphoreType.DMA,
        ],
        compiler_params=pltpu.CompilerParams(needs_layout_passes=False),
    )(_sc_body)
    return f(key, rm)


def kernel(x, W_enc, b_enc):
    xt = jnp.transpose(x, (1, 0, 2))
    b2 = b_enc.reshape(1, _DSAE)
    key, rm = _matmul_stage(xt, W_enc, b2)
    return _topk_mask_stage(key, rm)
